# baseline (speedup 1.0000x reference)
import functools
import math

import jax
import jax.numpy as jnp
from jax.experimental import pallas as pl
from jax.experimental.pallas import tpu as pltpu

_POOL_RATIO = 0.5
_BLK = 512


def _rup(v, m):
    return ((v + m - 1) // m) * m


def _mm_kernel(a_ref, b_ref, o_ref, ot_ref=None, *, bm, bn, valid, contract_t,
               emit_t):
    @pl.when(pl.program_id(2) == 0)
    def _():
        o_ref[...] = jnp.zeros_like(o_ref)

    a = a_ref[...].astype(jnp.bfloat16)
    b = b_ref[...].astype(jnp.bfloat16)
    if contract_t:
        o_ref[...] += jax.lax.dot_general(
            a, b, (((1,), (1,)), ((), ())),
            preferred_element_type=jnp.float32)
    else:
        o_ref[...] += jnp.dot(a, b, preferred_element_type=jnp.float32)

    @pl.when(pl.program_id(2) == pl.num_programs(2) - 1)
    def _():
        i = pl.program_id(0)
        j = pl.program_id(1)
        rows = i * bm + jax.lax.broadcasted_iota(jnp.int32, (bm, bn), 0)
        cols = j * bn + jax.lax.broadcasted_iota(jnp.int32, (bm, bn), 1)
        o = jnp.where((rows == cols) & (rows < valid), 1.0, o_ref[...])
        o_ref[...] = o
        if emit_t:
            ot_ref[...] = o.T


def _sq_restrict(a, b, *, bm, bn, bk, valid, contract_t=False, emit_t=False):
    m = a.shape[0]
    n = b.shape[0] if contract_t else b.shape[1]
    k = a.shape[1]
    grid = (m // bm, n // bn, k // bk)
    out_shape = [jax.ShapeDtypeStruct((m, n), jnp.float32)]
    out_specs = [pl.BlockSpec((bm, bn), lambda i, j, kk: (i, j))]
    if emit_t:
        out_shape.append(jax.ShapeDtypeStruct((n, m), jnp.float32))
        out_specs.append(pl.BlockSpec((bn, bm), lambda i, j, kk: (j, i)))
    if contract_t:
        b_spec = pl.BlockSpec((bn, bk), lambda i, j, kk: (j, kk))
    else:
        b_spec = pl.BlockSpec((bk, bn), lambda i, j, kk: (kk, j))
    res = pl.pallas_call(
        functools.partial(_mm_kernel, bm=bm, bn=bn, valid=valid,
                          contract_t=contract_t, emit_t=emit_t),
        grid=grid,
        in_specs=[pl.BlockSpec((bm, bk), lambda i, j, kk: (i, kk)), b_spec],
        out_specs=out_specs,
        out_shape=out_shape,
        compiler_params=pltpu.CompilerParams(
            dimension_semantics=("parallel", "parallel", "arbitrary")),
    )(a, b)
    return res if emit_t else res[0]


def _rowsum_kernel(a_ref, o_ref):
    @pl.when(pl.program_id(1) == 0)
    def _():
        o_ref[...] = jnp.zeros_like(o_ref)

    o_ref[...] += jnp.sum(a_ref[...], axis=1, keepdims=True)


def _rowsum(a, *, bm, bk):
    m, k = a.shape
    out = pl.pallas_call(
        _rowsum_kernel,
        grid=(m // bm, k // bk),
        in_specs=[pl.BlockSpec((bm, bk), lambda i, kk: (i, kk))],
        out_specs=pl.BlockSpec((bm, 1), lambda i, kk: (i, 0)),
        out_shape=jax.ShapeDtypeStruct((m, 1), jnp.float32),
        compiler_params=pltpu.CompilerParams(
            dimension_semantics=("parallel", "arbitrary")),
    )(a)
    return out[:, 0]


def _xw_kernel(x_ref, w_ref, dinv_ref, b_ref, o_ref, *, use_dinv, use_b):
    o = jnp.dot(x_ref[...], w_ref[...], preferred_element_type=jnp.float32)
    if use_dinv:
        o = o * dinv_ref[...]
    if use_b:
        o = o + b_ref[...]
    o_ref[...] = o


def _xw(x, w, dinv=None, b=None, *, bm):
    m, din = x.shape
    dout = w.shape[1]
    use_dinv = dinv is not None
    use_b = b is not None
    dinv_in = dinv[:, None] if use_dinv else jnp.zeros((m, 1), jnp.float32)
    b_in = b[None, :] if use_b else jnp.zeros((1, dout), jnp.float32)
    return pl.pallas_call(
        functools.partial(_xw_kernel, use_dinv=use_dinv, use_b=use_b),
        grid=(m // bm,),
        in_specs=[
            pl.BlockSpec((bm, din), lambda i: (i, 0)),
            pl.BlockSpec((din, dout), lambda i: (0, 0)),
            pl.BlockSpec((bm, 1), lambda i: (i, 0)),
            pl.BlockSpec((1, dout), lambda i: (0, 0)),
        ],
        out_specs=pl.BlockSpec((bm, dout), lambda i: (i, 0)),
        out_shape=jax.ShapeDtypeStruct((m, dout), jnp.float32),
        compiler_params=pltpu.CompilerParams(
            dimension_semantics=("parallel",)),
    )(x, w, dinv_in, b_in)


def _gcn_kernel(a_ref, z_ref, zs_ref, dinv_ref, s_ref, b_ref, o_ref, *, relu):
    @pl.when(pl.program_id(1) == 0)
    def _():
        o_ref[...] = s_ref[...] * zs_ref[...]

    o_ref[...] += jnp.dot(a_ref[...], z_ref[...],
                          preferred_element_type=jnp.float32)

    @pl.when(pl.program_id(1) == pl.num_programs(1) - 1)
    def _():
        o = dinv_ref[...] * o_ref[...] + b_ref[...]
        if relu:
            o = jnp.maximum(o, 0.0)
        o_ref[...] = o


def _gcn_conv(a, z, zscale, dinv, b, *, relu, bm, bk):
    m, d = z.shape
    return pl.pallas_call(
        functools.partial(_gcn_kernel, relu=relu),
        grid=(m // bm, m // bk),
        in_specs=[
            pl.BlockSpec((bm, bk), lambda i, kk: (i, kk)),
            pl.BlockSpec((bk, d), lambda i, kk: (kk, 0)),
            pl.BlockSpec((bm, d), lambda i, kk: (i, 0)),
            pl.BlockSpec((bm, 1), lambda i, kk: (i, 0)),
            pl.BlockSpec((bm, 1), lambda i, kk: (i, 0)),
            pl.BlockSpec((1, d), lambda i, kk: (0, 0)),
        ],
        out_specs=pl.BlockSpec((bm, d), lambda i, kk: (i, 0)),
        out_shape=jax.ShapeDtypeStruct((m, d), jnp.float32),
        compiler_params=pltpu.CompilerParams(
            dimension_semantics=("parallel", "arbitrary")),
    )(a, z, z, dinv[:, None], zscale[:, None], b[None, :])


def _pad_rows(a, m):
    return jnp.pad(a, ((0, m - a.shape[0]),) + ((0, 0),) * (a.ndim - 1))


def kernel(x, edge_index, W0, b0, W1, b1, W2, b2, p0, p1,
           Wu0, bu0, Wu1, bu1, Wd, bd):
    n, din = x.shape
    k0 = int(math.ceil(_POOL_RATIO * n))
    k1 = int(math.ceil(_POOL_RATIO * k0))
    np0, np1, np2 = _rup(n, _BLK), _rup(k0, _BLK), _rup(k1, _BLK)
    bmm = 1024 if (np0 % 1024 == 0 and np1 % 1024 == 0) else _BLK

    src = edge_index[0].astype(jnp.int32)
    dst = edge_index[1].astype(jnp.int32)
    selfe = (src == dst).astype(jnp.float32)
    nonself = 1.0 - selfe
    ones_e = jnp.ones_like(selfe)
    arn = jnp.arange(n, dtype=jnp.int32)

    indeg = jnp.zeros((n,), jnp.float32).at[dst].add(ones_e)
    c = jnp.zeros((n,), jnp.float32).at[dst].add(selfe)
    dinv0 = jax.lax.rsqrt(indeg + 2.0)
    dinv0p = _pad_rows(dinv0, np0)
    s0p = _pad_rows(c + 1.0, np0)

    Ms = jnp.zeros((np0, np0), jnp.float32).at[
        jnp.concatenate([dst, arn]), jnp.concatenate([src, arn])
    ].add(jnp.concatenate([nonself, jnp.ones((n,), jnp.float32)]),
          mode='drop')

    z0 = _xw(_pad_rows(x, np0), W0, dinv0p, bm=_BLK)
    x0 = _gcn_conv(Ms, z0, s0p, dinv0p, b0, relu=True, bm=_BLK, bk=_BLK)

    score0 = jnp.tanh((x0[:n] @ p0) / jnp.linalg.norm(p0))
    _, perm0 = jax.lax.top_k(score0, k0)
    xp1 = x0[:n][perm0] * score0[perm0][:, None]

    rank0 = jnp.full((n,), -1, jnp.int32).at[perm0].set(
        jnp.arange(k0, dtype=jnp.int32))
    rs = rank0[src]
    rd = rank0[dst]
    okG = (rs >= 0) & (src != dst)
    okH = (rd >= 0) & (src != dst)
    H = jnp.zeros((np1, np0), jnp.float32).at[
        jnp.concatenate([jnp.where(okH, rd, np1),
                         jnp.arange(k0, dtype=jnp.int32)]),
        jnp.concatenate([src, perm0]),
    ].add(jnp.concatenate([ones_e, jnp.ones((k0,), jnp.float32)]),
          mode='drop')
    G = jnp.zeros((np0, np1), jnp.float32).at[
        jnp.concatenate([dst, perm0]),
        jnp.concatenate([jnp.where(okG, rs, np1),
                         jnp.arange(k0, dtype=jnp.int32)]),
    ].add(jnp.concatenate([ones_e, jnp.ones((k0,), jnp.float32)]),
          mode='drop')
    M1S, M1St = _sq_restrict(H, G, bm=bmm, bn=bmm, bk=_BLK, valid=k0,
                             emit_t=True)

    deg1 = _rowsum(M1S, bm=_BLK, bk=_BLK)[:k0] + 1.0
    dinv1 = jax.lax.rsqrt(deg1)
    dinv1p = _pad_rows(dinv1, np1)
    ones1 = jnp.ones((np1,), jnp.float32)

    z1 = _xw(_pad_rows(xp1, np1), W1, dinv1p, bm=_BLK)
    x1 = _gcn_conv(M1S, z1, ones1, dinv1p, b1, relu=True, bm=_BLK, bk=_BLK)

    score1 = jnp.tanh((x1[:k0] @ p1) / jnp.linalg.norm(p1))
    _, perm1 = jax.lax.top_k(score1, k1)
    xp2 = x1[:k0][perm1] * score1[perm1][:, None]

    idxp1 = jnp.concatenate(
        [perm1, jnp.full((np2 - k1,), np1 - 1, jnp.int32)])
    H2 = M1S[idxp1]
    G2r = M1St[idxp1]
    M2S = _sq_restrict(H2, G2r, bm=_BLK, bn=_BLK, bk=_BLK, valid=k1,
                       contract_t=True)

    deg2 = _rowsum(M2S, bm=_BLK, bk=_BLK)[:k1] + 1.0
    dinv2 = jax.lax.rsqrt(deg2)
    dinv2p = _pad_rows(dinv2, np2)
    ones2 = jnp.ones((np2,), jnp.float32)

    z2 = _xw(_pad_rows(xp2, np2), W2, dinv2p, bm=_BLK)
    x2 = _gcn_conv(M2S, z2, ones2, dinv2p, b2, relu=True, bm=_BLK, bk=_BLK)

    up1 = jnp.zeros((k0, 128), jnp.float32).at[perm1].set(x2[:k1])
    cat1 = _pad_rows(jnp.concatenate([x1[:k0], up1], axis=1), np1)
    zu0 = _xw(cat1, Wu0, dinv1p, bm=_BLK)
    xu0 = _gcn_conv(M1S, zu0, ones1, dinv1p, bu0, relu=True, bm=_BLK,
                    bk=_BLK)

    up0 = jnp.zeros((n, 128), jnp.float32).at[perm0].set(xu0[:k0])
    cat0 = _pad_rows(jnp.concatenate([x0[:n], up0], axis=1), np0)
    zu1 = _xw(cat0, Wu1, dinv0p, bm=_BLK)
    xf = _gcn_conv(Ms, zu1, s0p, dinv0p, bu1, relu=False, bm=_BLK, bk=_BLK)

    ncls = Wd.shape[1]
    ncp = _rup(ncls, 128)
    out = _xw(xf, jnp.pad(Wd, ((0, 0), (0, ncp - ncls))), None,
              jnp.pad(bd, (0, ncp - ncls)), bm=_BLK)
    return out[:n, :ncls]


# speedup vs baseline: 1.4033x; 1.4033x over previous
import functools
import math

import jax
import jax.numpy as jnp
from jax.experimental import pallas as pl
from jax.experimental.pallas import tpu as pltpu

_POOL_RATIO = 0.5
_BLK = 512


def _rup(v, m):
    return ((v + m - 1) // m) * m


def _mm_kernel(a_ref, b_ref, o_ref, ot_ref=None, *, bm, bn, valid, contract_t,
               emit_t):
    @pl.when(pl.program_id(2) == 0)
    def _():
        o_ref[...] = jnp.zeros_like(o_ref)

    a = a_ref[...].astype(jnp.bfloat16)
    b = b_ref[...].astype(jnp.bfloat16)
    if contract_t:
        o_ref[...] += jax.lax.dot_general(
            a, b, (((1,), (1,)), ((), ())),
            preferred_element_type=jnp.float32)
    else:
        o_ref[...] += jnp.dot(a, b, preferred_element_type=jnp.float32)

    @pl.when(pl.program_id(2) == pl.num_programs(2) - 1)
    def _():
        i = pl.program_id(0)
        j = pl.program_id(1)
        rows = i * bm + jax.lax.broadcasted_iota(jnp.int32, (bm, bn), 0)
        cols = j * bn + jax.lax.broadcasted_iota(jnp.int32, (bm, bn), 1)
        o = jnp.where((rows == cols) & (rows < valid), 1.0, o_ref[...])
        o_ref[...] = o
        if emit_t:
            ot_ref[...] = o.T


def _sq_restrict(a, b, *, bm, bn, bk, valid, contract_t=False, emit_t=False):
    m = a.shape[0]
    n = b.shape[0] if contract_t else b.shape[1]
    k = a.shape[1]
    grid = (m // bm, n // bn, k // bk)
    out_shape = [jax.ShapeDtypeStruct((m, n), jnp.float32)]
    out_specs = [pl.BlockSpec((bm, bn), lambda i, j, kk: (i, j))]
    if emit_t:
        out_shape.append(jax.ShapeDtypeStruct((n, m), jnp.float32))
        out_specs.append(pl.BlockSpec((bn, bm), lambda i, j, kk: (j, i)))
    if contract_t:
        b_spec = pl.BlockSpec((bn, bk), lambda i, j, kk: (j, kk))
    else:
        b_spec = pl.BlockSpec((bk, bn), lambda i, j, kk: (kk, j))
    res = pl.pallas_call(
        functools.partial(_mm_kernel, bm=bm, bn=bn, valid=valid,
                          contract_t=contract_t, emit_t=emit_t),
        grid=grid,
        in_specs=[pl.BlockSpec((bm, bk), lambda i, j, kk: (i, kk)), b_spec],
        out_specs=out_specs,
        out_shape=out_shape,
        compiler_params=pltpu.CompilerParams(
            dimension_semantics=("parallel", "parallel", "arbitrary")),
    )(a, b)
    return res if emit_t else res[0]


def _rowsum_kernel(a_ref, o_ref):
    @pl.when(pl.program_id(1) == 0)
    def _():
        o_ref[...] = jnp.zeros_like(o_ref)

    o_ref[...] += jnp.sum(a_ref[...], axis=1, keepdims=True)


def _rowsum(a, *, bm, bk):
    m, k = a.shape
    out = pl.pallas_call(
        _rowsum_kernel,
        grid=(m // bm, k // bk),
        in_specs=[pl.BlockSpec((bm, bk), lambda i, kk: (i, kk))],
        out_specs=pl.BlockSpec((bm, 1), lambda i, kk: (i, 0)),
        out_shape=jax.ShapeDtypeStruct((m, 1), jnp.float32),
        compiler_params=pltpu.CompilerParams(
            dimension_semantics=("parallel", "arbitrary")),
    )(a)
    return out[:, 0]


def _xw_kernel(x_ref, w_ref, dinv_ref, b_ref, o_ref, *, use_dinv, use_b):
    o = jnp.dot(x_ref[...], w_ref[...], preferred_element_type=jnp.float32)
    if use_dinv:
        o = o * dinv_ref[...]
    if use_b:
        o = o + b_ref[...]
    o_ref[...] = o


def _xw(x, w, dinv=None, b=None, *, bm):
    m, din = x.shape
    dout = w.shape[1]
    use_dinv = dinv is not None
    use_b = b is not None
    dinv_in = dinv[:, None] if use_dinv else jnp.zeros((m, 1), jnp.float32)
    b_in = b[None, :] if use_b else jnp.zeros((1, dout), jnp.float32)
    return pl.pallas_call(
        functools.partial(_xw_kernel, use_dinv=use_dinv, use_b=use_b),
        grid=(m // bm,),
        in_specs=[
            pl.BlockSpec((bm, din), lambda i: (i, 0)),
            pl.BlockSpec((din, dout), lambda i: (0, 0)),
            pl.BlockSpec((bm, 1), lambda i: (i, 0)),
            pl.BlockSpec((1, dout), lambda i: (0, 0)),
        ],
        out_specs=pl.BlockSpec((bm, dout), lambda i: (i, 0)),
        out_shape=jax.ShapeDtypeStruct((m, dout), jnp.float32),
        compiler_params=pltpu.CompilerParams(
            dimension_semantics=("parallel",)),
    )(x, w, dinv_in, b_in)


def _gcn_kernel(a_ref, z_ref, zs_ref, dinv_ref, s_ref, b_ref, o_ref, *, relu):
    @pl.when(pl.program_id(1) == 0)
    def _():
        o_ref[...] = s_ref[...] * zs_ref[...]

    o_ref[...] += jnp.dot(a_ref[...], z_ref[...],
                          preferred_element_type=jnp.float32)

    @pl.when(pl.program_id(1) == pl.num_programs(1) - 1)
    def _():
        o = dinv_ref[...] * o_ref[...] + b_ref[...]
        if relu:
            o = jnp.maximum(o, 0.0)
        o_ref[...] = o


def _gcn_conv(a, z, zscale, dinv, b, *, relu, bm, bk):
    m, d = z.shape
    return pl.pallas_call(
        functools.partial(_gcn_kernel, relu=relu),
        grid=(m // bm, m // bk),
        in_specs=[
            pl.BlockSpec((bm, bk), lambda i, kk: (i, kk)),
            pl.BlockSpec((bk, d), lambda i, kk: (kk, 0)),
            pl.BlockSpec((bm, d), lambda i, kk: (i, 0)),
            pl.BlockSpec((bm, 1), lambda i, kk: (i, 0)),
            pl.BlockSpec((bm, 1), lambda i, kk: (i, 0)),
            pl.BlockSpec((1, d), lambda i, kk: (0, 0)),
        ],
        out_specs=pl.BlockSpec((bm, d), lambda i, kk: (i, 0)),
        out_shape=jax.ShapeDtypeStruct((m, d), jnp.float32),
        compiler_params=pltpu.CompilerParams(
            dimension_semantics=("parallel", "arbitrary")),
    )(a, z, z, dinv[:, None], zscale[:, None], b[None, :])


def _pad_rows(a, m):
    return jnp.pad(a, ((0, m - a.shape[0]),) + ((0, 0),) * (a.ndim - 1))


def kernel(x, edge_index, W0, b0, W1, b1, W2, b2, p0, p1,
           Wu0, bu0, Wu1, bu1, Wd, bd):
    n, din = x.shape
    k0 = int(math.ceil(_POOL_RATIO * n))
    k1 = int(math.ceil(_POOL_RATIO * k0))
    np0, np1, np2 = _rup(n, _BLK), _rup(k0, _BLK), _rup(k1, _BLK)
    bmm = 1024 if (np0 % 1024 == 0 and np1 % 1024 == 0) else _BLK

    src = edge_index[0].astype(jnp.int32)
    dst = edge_index[1].astype(jnp.int32)
    selfe = (src == dst).astype(jnp.float32)
    nonself = 1.0 - selfe
    ones_e = jnp.ones_like(selfe)
    arn = jnp.arange(n, dtype=jnp.int32)

    indeg = jnp.zeros((n,), jnp.float32).at[dst].add(ones_e)
    c = jnp.zeros((n,), jnp.float32).at[dst].add(selfe)
    dinv0 = jax.lax.rsqrt(indeg + 2.0)
    dinv0p = _pad_rows(dinv0, np0)
    s0p = _pad_rows(c + 1.0, np0)

    Ms = jnp.zeros((np0, np0), jnp.float32).at[
        jnp.concatenate([dst, arn]), jnp.concatenate([src, arn])
    ].add(jnp.concatenate([nonself, jnp.ones((n,), jnp.float32)]),
          mode='drop')

    z0 = _xw(_pad_rows(x, np0), W0, dinv0p, bm=_BLK)
    x0 = _gcn_conv(Ms, z0, s0p, dinv0p, b0, relu=True, bm=_BLK, bk=_BLK)

    score0 = jnp.tanh((x0[:n] @ p0) / jnp.linalg.norm(p0))
    _, perm0 = jax.lax.top_k(score0, k0)
    xp1 = x0[:n][perm0] * score0[perm0][:, None]

    rank0 = jnp.full((n,), -1, jnp.int32).at[perm0].set(
        jnp.arange(k0, dtype=jnp.int32))
    rs = rank0[src]
    okG = (rs >= 0) & (src != dst)
    idxp0 = jnp.concatenate(
        [perm0, jnp.full((np1 - k0,), np0 - 1, jnp.int32)])
    H = Ms[idxp0]
    G = jnp.zeros((np0, np1), jnp.float32).at[
        jnp.concatenate([dst, perm0]),
        jnp.concatenate([jnp.where(okG, rs, np1),
                         jnp.arange(k0, dtype=jnp.int32)]),
    ].add(jnp.concatenate([ones_e, jnp.ones((k0,), jnp.float32)]),
          mode='drop')
    M1S, M1St = _sq_restrict(H, G, bm=bmm, bn=bmm, bk=_BLK, valid=k0,
                             emit_t=True)

    deg1 = _rowsum(M1S, bm=_BLK, bk=_BLK)[:k0] + 1.0
    dinv1 = jax.lax.rsqrt(deg1)
    dinv1p = _pad_rows(dinv1, np1)
    ones1 = jnp.ones((np1,), jnp.float32)

    z1 = _xw(_pad_rows(xp1, np1), W1, dinv1p, bm=_BLK)
    x1 = _gcn_conv(M1S, z1, ones1, dinv1p, b1, relu=True, bm=_BLK, bk=_BLK)

    score1 = jnp.tanh((x1[:k0] @ p1) / jnp.linalg.norm(p1))
    _, perm1 = jax.lax.top_k(score1, k1)
    xp2 = x1[:k0][perm1] * score1[perm1][:, None]

    idxp1 = jnp.concatenate(
        [perm1, jnp.full((np2 - k1,), np1 - 1, jnp.int32)])
    H2 = M1S[idxp1]
    G2r = M1St[idxp1]
    M2S = _sq_restrict(H2, G2r, bm=_BLK, bn=_BLK, bk=_BLK, valid=k1,
                       contract_t=True)

    deg2 = _rowsum(M2S, bm=_BLK, bk=_BLK)[:k1] + 1.0
    dinv2 = jax.lax.rsqrt(deg2)
    dinv2p = _pad_rows(dinv2, np2)
    ones2 = jnp.ones((np2,), jnp.float32)

    z2 = _xw(_pad_rows(xp2, np2), W2, dinv2p, bm=_BLK)
    x2 = _gcn_conv(M2S, z2, ones2, dinv2p, b2, relu=True, bm=_BLK, bk=_BLK)

    up1 = jnp.zeros((k0, 128), jnp.float32).at[perm1].set(x2[:k1])
    cat1 = _pad_rows(jnp.concatenate([x1[:k0], up1], axis=1), np1)
    zu0 = _xw(cat1, Wu0, dinv1p, bm=_BLK)
    xu0 = _gcn_conv(M1S, zu0, ones1, dinv1p, bu0, relu=True, bm=_BLK,
                    bk=_BLK)

    up0 = jnp.zeros((n, 128), jnp.float32).at[perm0].set(xu0[:k0])
    cat0 = _pad_rows(jnp.concatenate([x0[:n], up0], axis=1), np0)
    zu1 = _xw(cat0, Wu1, dinv0p, bm=_BLK)
    xf = _gcn_conv(Ms, zu1, s0p, dinv0p, bu1, relu=False, bm=_BLK, bk=_BLK)

    ncls = Wd.shape[1]
    ncp = _rup(ncls, 128)
    out = _xw(xf, jnp.pad(Wd, ((0, 0), (0, ncp - ncls))), None,
              jnp.pad(bd, (0, ncp - ncls)), bm=_BLK)
    return out[:n, :ncls]


# speedup vs baseline: 1.8059x; 1.2869x over previous
import functools
import math

import jax
import jax.numpy as jnp
from jax.experimental import pallas as pl
from jax.experimental.pallas import tpu as pltpu

_POOL_RATIO = 0.5
_BLK = 512


def _rup(v, m):
    return ((v + m - 1) // m) * m


def _mm_kernel(a_ref, b_ref, o_ref, ot_ref=None, *, bm, bn, valid, contract_t,
               emit_t):
    @pl.when(pl.program_id(2) == 0)
    def _():
        o_ref[...] = jnp.zeros_like(o_ref)

    a = a_ref[...].astype(jnp.bfloat16)
    b = b_ref[...].astype(jnp.bfloat16)
    if contract_t:
        o_ref[...] += jax.lax.dot_general(
            a, b, (((1,), (1,)), ((), ())),
            preferred_element_type=jnp.float32)
    else:
        o_ref[...] += jnp.dot(a, b, preferred_element_type=jnp.float32)

    @pl.when(pl.program_id(2) == pl.num_programs(2) - 1)
    def _():
        i = pl.program_id(0)
        j = pl.program_id(1)
        rows = i * bm + jax.lax.broadcasted_iota(jnp.int32, (bm, bn), 0)
        cols = j * bn + jax.lax.broadcasted_iota(jnp.int32, (bm, bn), 1)
        o = jnp.where((rows == cols) & (rows < valid), 1.0, o_ref[...])
        o_ref[...] = o
        if emit_t:
            ot_ref[...] = o.T


def _sq_restrict(a, b, *, bm, bn, bk, valid, contract_t=False, emit_t=False):
    m = a.shape[0]
    n = b.shape[0] if contract_t else b.shape[1]
    k = a.shape[1]
    grid = (m // bm, n // bn, k // bk)
    out_shape = [jax.ShapeDtypeStruct((m, n), jnp.float32)]
    out_specs = [pl.BlockSpec((bm, bn), lambda i, j, kk: (i, j))]
    if emit_t:
        out_shape.append(jax.ShapeDtypeStruct((n, m), jnp.float32))
        out_specs.append(pl.BlockSpec((bn, bm), lambda i, j, kk: (j, i)))
    if contract_t:
        b_spec = pl.BlockSpec((bn, bk), lambda i, j, kk: (j, kk))
    else:
        b_spec = pl.BlockSpec((bk, bn), lambda i, j, kk: (kk, j))
    res = pl.pallas_call(
        functools.partial(_mm_kernel, bm=bm, bn=bn, valid=valid,
                          contract_t=contract_t, emit_t=emit_t),
        grid=grid,
        in_specs=[pl.BlockSpec((bm, bk), lambda i, j, kk: (i, kk)), b_spec],
        out_specs=out_specs,
        out_shape=out_shape,
        compiler_params=pltpu.CompilerParams(
            dimension_semantics=("parallel", "parallel", "arbitrary")),
    )(a, b)
    return res if emit_t else res[0]


def _t_kernel(a_ref, o_ref):
    o_ref[...] = a_ref[...].T


def _transpose(a, *, blk):
    m, k = a.shape
    return pl.pallas_call(
        _t_kernel,
        grid=(m // blk, k // blk),
        in_specs=[pl.BlockSpec((blk, blk), lambda i, j: (i, j))],
        out_specs=pl.BlockSpec((blk, blk), lambda i, j: (j, i)),
        out_shape=jax.ShapeDtypeStruct((k, m), jnp.float32),
        compiler_params=pltpu.CompilerParams(
            dimension_semantics=("parallel", "parallel")),
    )(a)


def _rowsum_kernel(a_ref, o_ref):
    @pl.when(pl.program_id(1) == 0)
    def _():
        o_ref[...] = jnp.zeros_like(o_ref)

    o_ref[...] += jnp.sum(a_ref[...], axis=1, keepdims=True)


def _rowsum(a, *, bm, bk):
    m, k = a.shape
    out = pl.pallas_call(
        _rowsum_kernel,
        grid=(m // bm, k // bk),
        in_specs=[pl.BlockSpec((bm, bk), lambda i, kk: (i, kk))],
        out_specs=pl.BlockSpec((bm, 1), lambda i, kk: (i, 0)),
        out_shape=jax.ShapeDtypeStruct((m, 1), jnp.float32),
        compiler_params=pltpu.CompilerParams(
            dimension_semantics=("parallel", "arbitrary")),
    )(a)
    return out[:, 0]


def _xw_kernel(x_ref, w_ref, dinv_ref, b_ref, o_ref, *, use_dinv, use_b):
    o = jnp.dot(x_ref[...], w_ref[...], preferred_element_type=jnp.float32)
    if use_dinv:
        o = o * dinv_ref[...]
    if use_b:
        o = o + b_ref[...]
    o_ref[...] = o


def _xw(x, w, dinv=None, b=None, *, bm):
    m, din = x.shape
    dout = w.shape[1]
    use_dinv = dinv is not None
    use_b = b is not None
    dinv_in = dinv[:, None] if use_dinv else jnp.zeros((m, 1), jnp.float32)
    b_in = b[None, :] if use_b else jnp.zeros((1, dout), jnp.float32)
    return pl.pallas_call(
        functools.partial(_xw_kernel, use_dinv=use_dinv, use_b=use_b),
        grid=(m // bm,),
        in_specs=[
            pl.BlockSpec((bm, din), lambda i: (i, 0)),
            pl.BlockSpec((din, dout), lambda i: (0, 0)),
            pl.BlockSpec((bm, 1), lambda i: (i, 0)),
            pl.BlockSpec((1, dout), lambda i: (0, 0)),
        ],
        out_specs=pl.BlockSpec((bm, dout), lambda i: (i, 0)),
        out_shape=jax.ShapeDtypeStruct((m, dout), jnp.float32),
        compiler_params=pltpu.CompilerParams(
            dimension_semantics=("parallel",)),
    )(x, w, dinv_in, b_in)


def _gcn_kernel(a_ref, z_ref, zs_ref, dinv_ref, s_ref, b_ref, o_ref, *, relu):
    @pl.when(pl.program_id(1) == 0)
    def _():
        o_ref[...] = s_ref[...] * zs_ref[...]

    o_ref[...] += jnp.dot(a_ref[...], z_ref[...],
                          preferred_element_type=jnp.float32)

    @pl.when(pl.program_id(1) == pl.num_programs(1) - 1)
    def _():
        o = dinv_ref[...] * o_ref[...] + b_ref[...]
        if relu:
            o = jnp.maximum(o, 0.0)
        o_ref[...] = o


def _gcn_conv(a, z, zscale, dinv, b, *, relu, bm, bk):
    m, d = z.shape
    return pl.pallas_call(
        functools.partial(_gcn_kernel, relu=relu),
        grid=(m // bm, m // bk),
        in_specs=[
            pl.BlockSpec((bm, bk), lambda i, kk: (i, kk)),
            pl.BlockSpec((bk, d), lambda i, kk: (kk, 0)),
            pl.BlockSpec((bm, d), lambda i, kk: (i, 0)),
            pl.BlockSpec((bm, 1), lambda i, kk: (i, 0)),
            pl.BlockSpec((bm, 1), lambda i, kk: (i, 0)),
            pl.BlockSpec((1, d), lambda i, kk: (0, 0)),
        ],
        out_specs=pl.BlockSpec((bm, d), lambda i, kk: (i, 0)),
        out_shape=jax.ShapeDtypeStruct((m, d), jnp.float32),
        compiler_params=pltpu.CompilerParams(
            dimension_semantics=("parallel", "arbitrary")),
    )(a, z, z, dinv[:, None], zscale[:, None], b[None, :])


def _pad_rows(a, m):
    return jnp.pad(a, ((0, m - a.shape[0]),) + ((0, 0),) * (a.ndim - 1))


def kernel(x, edge_index, W0, b0, W1, b1, W2, b2, p0, p1,
           Wu0, bu0, Wu1, bu1, Wd, bd):
    n, din = x.shape
    k0 = int(math.ceil(_POOL_RATIO * n))
    k1 = int(math.ceil(_POOL_RATIO * k0))
    np0, np1, np2 = _rup(n, _BLK), _rup(k0, _BLK), _rup(k1, _BLK)
    bmm = 1024 if (np0 % 1024 == 0 and np1 % 1024 == 0) else _BLK

    src = edge_index[0].astype(jnp.int32)
    dst = edge_index[1].astype(jnp.int32)
    selfe = (src == dst).astype(jnp.float32)
    nonself = 1.0 - selfe
    ones_e = jnp.ones_like(selfe)
    arn = jnp.arange(n, dtype=jnp.int32)

    indeg = jnp.zeros((n,), jnp.float32).at[dst].add(ones_e)
    c = jnp.zeros((n,), jnp.float32).at[dst].add(selfe)
    dinv0 = jax.lax.rsqrt(indeg + 2.0)
    dinv0p = _pad_rows(dinv0, np0)
    s0p = _pad_rows(c + 1.0, np0)

    Ms = jnp.zeros((np0, np0), jnp.float32).at[
        jnp.concatenate([dst, arn]), jnp.concatenate([src, arn])
    ].add(jnp.concatenate([nonself, jnp.ones((n,), jnp.float32)]),
          mode='drop')

    z0 = _xw(_pad_rows(x, np0), W0, dinv0p, bm=_BLK)
    x0 = _gcn_conv(Ms, z0, s0p, dinv0p, b0, relu=True, bm=_BLK, bk=_BLK)

    score0 = jnp.tanh((x0[:n] @ p0) / jnp.linalg.norm(p0))
    _, perm0 = jax.lax.top_k(score0, k0)
    xp1 = x0[:n][perm0] * score0[perm0][:, None]

    idxp0 = jnp.concatenate(
        [perm0, jnp.full((np1 - k0,), np0 - 1, jnp.int32)])
    H = Ms[idxp0]
    Mst = _transpose(Ms, blk=_BLK)
    Gr = Mst[idxp0]
    M1S, M1St = _sq_restrict(H, Gr, bm=bmm, bn=bmm, bk=_BLK, valid=k0,
                             contract_t=True, emit_t=True)

    deg1 = _rowsum(M1S, bm=_BLK, bk=_BLK)[:k0] + 1.0
    dinv1 = jax.lax.rsqrt(deg1)
    dinv1p = _pad_rows(dinv1, np1)
    ones1 = jnp.ones((np1,), jnp.float32)

    z1 = _xw(_pad_rows(xp1, np1), W1, dinv1p, bm=_BLK)
    x1 = _gcn_conv(M1S, z1, ones1, dinv1p, b1, relu=True, bm=_BLK, bk=_BLK)

    score1 = jnp.tanh((x1[:k0] @ p1) / jnp.linalg.norm(p1))
    _, perm1 = jax.lax.top_k(score1, k1)
    xp2 = x1[:k0][perm1] * score1[perm1][:, None]

    idxp1 = jnp.concatenate(
        [perm1, jnp.full((np2 - k1,), np1 - 1, jnp.int32)])
    H2 = M1S[idxp1]
    G2r = M1St[idxp1]
    M2S = _sq_restrict(H2, G2r, bm=_BLK, bn=_BLK, bk=_BLK, valid=k1,
                       contract_t=True)

    deg2 = _rowsum(M2S, bm=_BLK, bk=_BLK)[:k1] + 1.0
    dinv2 = jax.lax.rsqrt(deg2)
    dinv2p = _pad_rows(dinv2, np2)
    ones2 = jnp.ones((np2,), jnp.float32)

    z2 = _xw(_pad_rows(xp2, np2), W2, dinv2p, bm=_BLK)
    x2 = _gcn_conv(M2S, z2, ones2, dinv2p, b2, relu=True, bm=_BLK, bk=_BLK)

    up1 = jnp.zeros((k0, 128), jnp.float32).at[perm1].set(x2[:k1])
    cat1 = _pad_rows(jnp.concatenate([x1[:k0], up1], axis=1), np1)
    zu0 = _xw(cat1, Wu0, dinv1p, bm=_BLK)
    xu0 = _gcn_conv(M1S, zu0, ones1, dinv1p, bu0, relu=True, bm=_BLK,
                    bk=_BLK)

    up0 = jnp.zeros((n, 128), jnp.float32).at[perm0].set(xu0[:k0])
    cat0 = _pad_rows(jnp.concatenate([x0[:n], up0], axis=1), np0)
    zu1 = _xw(cat0, Wu1, dinv0p, bm=_BLK)
    xf = _gcn_conv(Ms, zu1, s0p, dinv0p, bu1, relu=False, bm=_BLK, bk=_BLK)

    ncls = Wd.shape[1]
    ncp = _rup(ncls, 128)
    out = _xw(xf, jnp.pad(Wd, ((0, 0), (0, ncp - ncls))), None,
              jnp.pad(bd, (0, ncp - ncls)), bm=_BLK)
    return out[:n, :ncls]
